# baseline (speedup 1.0000x reference)
import functools

import jax
import jax.numpy as jnp
from jax import lax
from jax.experimental import pallas as pl
from jax.experimental.pallas import tpu as pltpu
from jax.experimental.pallas import tpu_sc as plsc

NC = 2
NS = 16
NW = NC * NS
L = 16

R_MAX = 5.0
SUB = 128
CHUNK = 2000
NSUB = CHUNK // SUB
GROUPS = CHUNK // L

PI = 3.14159265358979
TAB_W = 4

S3, S5, S7, S9 = -1.0 / 6.0, 1.0 / 120.0, -1.0 / 5040.0, 1.0 / 362880.0
C2, C4, C6, C8 = -0.5, 1.0 / 24.0, -1.0 / 720.0, 1.0 / 40320.0


@functools.lru_cache(maxsize=None)
def _build(e_pad, cpw):
    mesh = plsc.VectorSubcoreMesh(
        core_axis_name="c", subcore_axis_name="s", num_cores=NC, num_subcores=NS
    )

    @functools.partial(
        pl.kernel,
        out_type=[jax.ShapeDtypeStruct((e_pad,), jnp.float32)] * 9,
        mesh=mesh,
        scratch_types=[
            pltpu.VMEM((CHUNK,), jnp.int32),
            pltpu.VMEM((CHUNK,), jnp.int32),
            pltpu.VMEM((CHUNK, TAB_W), jnp.float32),
            pltpu.VMEM((CHUNK, TAB_W), jnp.float32),
            [pltpu.VMEM((CHUNK,), jnp.float32) for _ in range(9)],
            pltpu.SemaphoreType.DMA,
            pltpu.SemaphoreType.DMA,
        ],
        compiler_params=pltpu.CompilerParams(needs_layout_passes=False, use_tc_tiling_on_sc=False),
    )
    def edge_kernel(tab_h, src_h, trg_h, *rest):
        outs = rest[:9]
        sidx, tidx, srow, trow, obuf, sem_s, sem_t = rest[9:]
        wid = lax.axis_index("s") * NC + lax.axis_index("c")
        base = wid * (cpw * CHUNK)

        def do_chunk(g, carry):
            cbase = base + g * CHUNK
            pltpu.sync_copy(src_h.at[pl.ds(cbase, CHUNK)], sidx)
            pltpu.sync_copy(trg_h.at[pl.ds(cbase, CHUNK)], tidx)
            cps = [
                pltpu.async_copy(tab_h.at[sidx], srow, sem_s),
                pltpu.async_copy(tab_h.at[tidx], trow, sem_t),
            ]
            for cp in cps:
                cp.wait()

            def do_group(i, carry2):
                sl = pl.ds(i * L, L)
                rows = i * L + lax.iota(jnp.int32, L)
                zero = lax.iota(jnp.int32, L) * 0
                sx = plsc.load_gather(srow, [rows, zero])
                sy = plsc.load_gather(srow, [rows, zero + 1])
                sz = plsc.load_gather(srow, [rows, zero + 2])
                sc = plsc.load_gather(srow, [rows, zero + 3])
                tx = plsc.load_gather(trow, [rows, zero])
                ty = plsc.load_gather(trow, [rows, zero + 1])
                tz = plsc.load_gather(trow, [rows, zero + 2])
                tc = plsc.load_gather(trow, [rows, zero + 3])

                dx = tx - sx
                dy = ty - sy
                dz = tz - sz
                d2 = dx * dx + dy * dy + dz * dz

                d2i = lax.bitcast_convert_type(d2, jnp.int32)
                y = lax.bitcast_convert_type(0x5F3759DF - (d2i >> 1), jnp.float32)
                y = y * (1.5 - 0.5 * d2 * y * y)
                y = y * (1.5 - 0.5 * d2 * y * y)
                y = y * (1.5 - 0.5 * d2 * y * y)
                r = d2 * y
                inv = y

                u = r * (1.0 / R_MAX)
                u2 = u * u
                u3 = u2 * u
                u6 = u3 * u3
                cut = 1.0 - 28.0 * u6 + 48.0 * u6 * u - 21.0 * u6 * u2
                cut = jnp.where(u < 1.0, cut, 0.0)

                th = jnp.minimum(u, 1.0) * PI
                flip = th > (0.5 * PI)
                thr = jnp.where(flip, PI - th, th)
                z = thr * thr
                s1 = thr * (1.0 + z * (S3 + z * (S5 + z * (S7 + z * S9))))
                c1 = 1.0 + z * (C2 + z * (C4 + z * (C6 + z * C8)))
                c1 = jnp.where(flip, -c1, c1)

                two_c = 2.0 * c1
                scale = (2.0 / R_MAX) * inv * cut
                s_prev = s1
                obuf[0][sl] = s1 * scale
                s_cur = two_c * s1
                obuf[1][sl] = s_cur * scale
                for k in range(2, 8):
                    s_next = two_c * s_cur - s_prev
                    s_prev, s_cur = s_cur, s_next
                    obuf[k][sl] = s_cur * scale
                mem = jnp.where(sc == tc, 1.0, 0.0)
                obuf[8][sl] = mem
                return carry2

            lax.fori_loop(0, GROUPS, do_group, 0)
            for k in range(9):
                pltpu.sync_copy(obuf[k], outs[k].at[pl.ds(cbase, CHUNK)])
            return carry

        lax.fori_loop(0, cpw, do_chunk, 0)

    return edge_kernel


def kernel(pos, edge_index, edge_class, bessel_weights):
    e = edge_index.shape[1]
    cpw = -(-e // (NW * CHUNK))
    e_pad = NW * cpw * CHUNK
    src = edge_index[0]
    trg = edge_index[1]
    if e_pad != e:
        zpad = jnp.zeros((e_pad - e,), jnp.int32)
        src = jnp.concatenate([src, zpad])
        trg = jnp.concatenate([trg, zpad])
    n = pos.shape[0]
    tab = jnp.concatenate(
        [pos, edge_class[:, None].astype(jnp.float32)], axis=1)
    outs = _build(e_pad, cpw)(tab, src, trg)
    return jnp.stack(outs, axis=1)[:e]


# speedup vs baseline: 1.2996x; 1.2996x over previous
import functools

import jax
import jax.numpy as jnp
from jax import lax
from jax.experimental import pallas as pl
from jax.experimental.pallas import tpu as pltpu
from jax.experimental.pallas import tpu_sc as plsc

NC = 2
NS = 16
NW = NC * NS
L = 16

R_MAX = 5.0
SUB = 128
CHUNK = 2000
NSUB = CHUNK // SUB
GROUPS = CHUNK // L

PI = 3.14159265358979
TAB_W = 8

S3, S5, S7, S9 = -1.0 / 6.0, 1.0 / 120.0, -1.0 / 5040.0, 1.0 / 362880.0
C2, C4, C6, C8 = -0.5, 1.0 / 24.0, -1.0 / 720.0, 1.0 / 40320.0


@functools.lru_cache(maxsize=None)
def _build(e_pad, cpw):
    mesh = plsc.VectorSubcoreMesh(
        core_axis_name="c", subcore_axis_name="s", num_cores=NC, num_subcores=NS
    )

    @functools.partial(
        pl.kernel,
        out_type=[jax.ShapeDtypeStruct((e_pad,), jnp.float32)] * 9,
        mesh=mesh,
        scratch_types=[
            pltpu.VMEM((CHUNK,), jnp.int32),
            pltpu.VMEM((CHUNK,), jnp.int32),
            pltpu.VMEM((CHUNK, TAB_W), jnp.float32),
            pltpu.VMEM((CHUNK, TAB_W), jnp.float32),
            [pltpu.VMEM((CHUNK,), jnp.float32) for _ in range(9)],
            pltpu.SemaphoreType.DMA,
            pltpu.SemaphoreType.DMA,
        ],
        compiler_params=pltpu.CompilerParams(needs_layout_passes=False, use_tc_tiling_on_sc=False),
    )
    def edge_kernel(tab_h, src_h, trg_h, *rest):
        outs = rest[:9]
        sidx, tidx, srow, trow, obuf, sem_s, sem_t = rest[9:]
        wid = lax.axis_index("s") * NC + lax.axis_index("c")
        base = wid * (cpw * CHUNK)

        def do_chunk(g, carry):
            cbase = base + g * CHUNK
            pltpu.sync_copy(src_h.at[pl.ds(cbase, CHUNK)], sidx)
            pltpu.sync_copy(trg_h.at[pl.ds(cbase, CHUNK)], tidx)
            cps = [
                pltpu.async_copy(tab_h.at[sidx], srow, sem_s),
                pltpu.async_copy(tab_h.at[tidx], trow, sem_t),
            ]
            for cp in cps:
                cp.wait()

            def do_group(i, carry2):
                sl = pl.ds(i * L, L)
                rows = i * L + lax.iota(jnp.int32, L)
                zero = lax.iota(jnp.int32, L) * 0
                sx = plsc.load_gather(srow, [rows, zero])
                sy = plsc.load_gather(srow, [rows, zero + 1])
                sz = plsc.load_gather(srow, [rows, zero + 2])
                sc = plsc.load_gather(srow, [rows, zero + 3])
                tx = plsc.load_gather(trow, [rows, zero])
                ty = plsc.load_gather(trow, [rows, zero + 1])
                tz = plsc.load_gather(trow, [rows, zero + 2])
                tc = plsc.load_gather(trow, [rows, zero + 3])

                dx = tx - sx
                dy = ty - sy
                dz = tz - sz
                d2 = dx * dx + dy * dy + dz * dz

                d2i = lax.bitcast_convert_type(d2, jnp.int32)
                y = lax.bitcast_convert_type(0x5F3759DF - (d2i >> 1), jnp.float32)
                y = y * (1.5 - 0.5 * d2 * y * y)
                y = y * (1.5 - 0.5 * d2 * y * y)
                y = y * (1.5 - 0.5 * d2 * y * y)
                r = d2 * y
                inv = y

                u = r * (1.0 / R_MAX)
                u2 = u * u
                u3 = u2 * u
                u6 = u3 * u3
                cut = 1.0 - 28.0 * u6 + 48.0 * u6 * u - 21.0 * u6 * u2
                cut = jnp.where(u < 1.0, cut, 0.0)

                th = jnp.minimum(u, 1.0) * PI
                flip = th > (0.5 * PI)
                thr = jnp.where(flip, PI - th, th)
                z = thr * thr
                s1 = thr * (1.0 + z * (S3 + z * (S5 + z * (S7 + z * S9))))
                c1 = 1.0 + z * (C2 + z * (C4 + z * (C6 + z * C8)))
                c1 = jnp.where(flip, -c1, c1)

                two_c = 2.0 * c1
                scale = (2.0 / R_MAX) * inv * cut
                s_prev = s1
                obuf[0][sl] = s1 * scale
                s_cur = two_c * s1
                obuf[1][sl] = s_cur * scale
                for k in range(2, 8):
                    s_next = two_c * s_cur - s_prev
                    s_prev, s_cur = s_cur, s_next
                    obuf[k][sl] = s_cur * scale
                mem = jnp.where(sc == tc, 1.0, 0.0)
                obuf[8][sl] = mem
                return carry2

            lax.fori_loop(0, GROUPS, do_group, 0)
            for k in range(9):
                pltpu.sync_copy(obuf[k], outs[k].at[pl.ds(cbase, CHUNK)])
            return carry

        lax.fori_loop(0, cpw, do_chunk, 0)

    return edge_kernel


def kernel(pos, edge_index, edge_class, bessel_weights):
    e = edge_index.shape[1]
    cpw = -(-e // (NW * CHUNK))
    e_pad = NW * cpw * CHUNK
    src = edge_index[0]
    trg = edge_index[1]
    if e_pad != e:
        zpad = jnp.zeros((e_pad - e,), jnp.int32)
        src = jnp.concatenate([src, zpad])
        trg = jnp.concatenate([trg, zpad])
    n = pos.shape[0]
    tab = jnp.concatenate(
        [pos, edge_class[:, None].astype(jnp.float32),
         jnp.zeros((n, 4), jnp.float32)], axis=1)
    cpw_a = (cpw + 1) // 2
    cpw_b = cpw - cpw_a
    e_a = NW * cpw_a * CHUNK
    if cpw_b == 0:
        return jnp.stack(_build(e_a, cpw_a)(tab, src, trg), axis=1)[:e]
    outs_a = _build(e_a, cpw_a)(tab, src[:e_a], trg[:e_a])
    outs_b = _build(e_pad - e_a, cpw_b)(tab, src[e_a:], trg[e_a:])
    stacked_a = jnp.stack(outs_a, axis=1)
    stacked_b = jnp.stack(outs_b, axis=1)
    return jnp.concatenate([stacked_a, stacked_b], axis=0)[:e]
